# baseline (speedup 1.0000x reference)
import jax
import jax.numpy as jnp
from jax.experimental import pallas as pl
from jax.experimental.pallas import tpu as pltpu

_NCH = 8


def _dot(a, b):
    return jnp.dot(a, b, preferred_element_type=jnp.float32)


def _dott(a, b):
    return jax.lax.dot_general(a, b, (((0,), (0,)), ((), ())),
                               preferred_element_type=jnp.float32)


def _gcn_body(x_ref, w1_ref, b1_ref, w2_ref, b2_ref, w3_ref, b3_ref,
              wc_ref, bc_ref, out_ref, h_ref, xn_ref, t1_ref, t2_ref, t3_ref):
    p = pl.program_id(0)
    c = pl.program_id(1)
    ch = x_ref.shape[0]

    @pl.when(jnp.logical_and(p == 0, c == 0))
    def _init():
        t1_ref[...] = jnp.zeros_like(t1_ref)
        t2_ref[...] = jnp.zeros_like(t2_ref)
        t3_ref[...] = jnp.zeros_like(t3_ref)

    @pl.when(p == 0)
    def _pass0():
        x = x_ref[...]
        norm = jnp.sqrt(jnp.sum(x * x, axis=1, keepdims=True))
        xn = x / jnp.maximum(norm, 1e-8)
        xn_ref[pl.ds(c * ch, ch), :] = xn
        t1_ref[...] += _dott(xn, _dot(x, w1_ref[...]))

    def _mid(t_in_ref, b_ref, w_next_ref, t_out_ref):
        xn = xn_ref[pl.ds(c * ch, ch), :]
        hh = _dot(xn, t_in_ref[...]) + b_ref[...]
        hh = jnp.where(hh >= 0, hh, 0.25 * hh)
        t_out_ref[...] += _dott(xn, _dot(hh, w_next_ref[...]))

    pl.when(p == 1)(lambda: _mid(t1_ref, b1_ref, w2_ref, t2_ref))
    pl.when(p == 2)(lambda: _mid(t2_ref, b2_ref, w3_ref, t3_ref))

    @pl.when(p == 3)
    def _pass3():
        xn = xn_ref[pl.ds(c * ch, ch), :]
        hh = _dot(xn, t3_ref[...]) + b3_ref[...]
        hh = jnp.where(hh >= 0, hh, 0.25 * hh)
        h_ref[...] = hh
        out_ref[...] = _dot(hh, wc_ref[...]) + bc_ref[...]


def kernel(x, W1, b1, W2, b2, W3, b3, Wc, bc):
    n, d = x.shape
    do = Wc.shape[1]
    ch = n // _NCH

    def _wspec(shape):
        return pl.BlockSpec(shape, lambda p, c: (0,) * len(shape))

    out, h = pl.pallas_call(
        _gcn_body,
        grid=(4, _NCH),
        in_specs=[
            pl.BlockSpec((ch, d), lambda p, c: (jnp.where(p == 0, c, _NCH - 1), 0)),
            _wspec((d, do)), _wspec((1, do)),
            _wspec((do, do)), _wspec((1, do)),
            _wspec((do, do)), _wspec((1, do)),
            _wspec((d, do)), _wspec((1, do)),
        ],
        out_specs=(
            pl.BlockSpec((ch, do), lambda p, c: (jnp.where(p == 3, c, 0), 0)),
            pl.BlockSpec((ch, do), lambda p, c: (jnp.where(p == 3, c, 0), 0)),
        ),
        out_shape=(
            jax.ShapeDtypeStruct((n, do), jnp.float32),
            jax.ShapeDtypeStruct((n, do), jnp.float32),
        ),
        scratch_shapes=[
            pltpu.VMEM((n, d), jnp.float32),
            pltpu.VMEM((do, do), jnp.float32),
            pltpu.VMEM((do, do), jnp.float32),
            pltpu.VMEM((do, do), jnp.float32),
        ],
    )(x, W1, b1[0, 0][None, :], W2, b2[0, 0][None, :],
      W3, b3[0, 0][None, :], Wc, bc[None, :])
    return (out, h)


# speedup vs baseline: 1.9455x; 1.9455x over previous
import jax
import jax.numpy as jnp
from jax.experimental import pallas as pl
from jax.experimental.pallas import tpu as pltpu

_NCH = 2


def _dot(a, b):
    return jnp.dot(a, b, preferred_element_type=jnp.float32)


def _dott(a, b):
    return jax.lax.dot_general(a, b, (((0,), (0,)), ((), ())),
                               preferred_element_type=jnp.float32)


def _gcn_body(x_ref, w1_ref, b1_ref, w2_ref, b2_ref, w3_ref, b3_ref,
              wc_ref, bc_ref, out_ref, h_ref, xn_ref, t1_ref, t2_ref, t3_ref):
    p = pl.program_id(0)
    c = pl.program_id(1)
    ch = x_ref.shape[0]

    @pl.when(jnp.logical_and(p == 0, c == 0))
    def _init():
        t1_ref[...] = jnp.zeros_like(t1_ref)
        t2_ref[...] = jnp.zeros_like(t2_ref)
        t3_ref[...] = jnp.zeros_like(t3_ref)

    @pl.when(p == 0)
    def _pass0():
        x = x_ref[...]
        norm = jnp.sqrt(jnp.sum(x * x, axis=1, keepdims=True))
        xn = x / jnp.maximum(norm, 1e-8)
        xn_ref[pl.ds(c * ch, ch), :] = xn
        t1_ref[...] += _dott(xn, _dot(x, w1_ref[...]))

    def _mid(t_in_ref, b_ref, w_next_ref, t_out_ref):
        xn = xn_ref[pl.ds(c * ch, ch), :]
        hh = _dot(xn, t_in_ref[...]) + b_ref[...]
        hh = jnp.where(hh >= 0, hh, 0.25 * hh)
        t_out_ref[...] += _dott(xn, _dot(hh, w_next_ref[...]))

    pl.when(p == 1)(lambda: _mid(t1_ref, b1_ref, w2_ref, t2_ref))
    pl.when(p == 2)(lambda: _mid(t2_ref, b2_ref, w3_ref, t3_ref))

    @pl.when(p == 3)
    def _pass3():
        xn = xn_ref[pl.ds(c * ch, ch), :]
        hh = _dot(xn, t3_ref[...]) + b3_ref[...]
        hh = jnp.where(hh >= 0, hh, 0.25 * hh)
        h_ref[...] = hh
        out_ref[...] = _dot(hh, wc_ref[...]) + bc_ref[...]


def kernel(x, W1, b1, W2, b2, W3, b3, Wc, bc):
    n, d = x.shape
    do = Wc.shape[1]
    ch = n // _NCH

    def _wspec(shape):
        return pl.BlockSpec(shape, lambda p, c: (0,) * len(shape))

    out, h = pl.pallas_call(
        _gcn_body,
        grid=(4, _NCH),
        in_specs=[
            pl.BlockSpec((ch, d), lambda p, c: (jnp.where(p == 0, c, _NCH - 1), 0)),
            _wspec((d, do)), _wspec((1, do)),
            _wspec((do, do)), _wspec((1, do)),
            _wspec((do, do)), _wspec((1, do)),
            _wspec((d, do)), _wspec((1, do)),
        ],
        out_specs=(
            pl.BlockSpec((ch, do), lambda p, c: (jnp.where(p == 3, c, 0), 0)),
            pl.BlockSpec((ch, do), lambda p, c: (jnp.where(p == 3, c, 0), 0)),
        ),
        out_shape=(
            jax.ShapeDtypeStruct((n, do), jnp.float32),
            jax.ShapeDtypeStruct((n, do), jnp.float32),
        ),
        scratch_shapes=[
            pltpu.VMEM((n, d), jnp.float32),
            pltpu.VMEM((do, do), jnp.float32),
            pltpu.VMEM((do, do), jnp.float32),
            pltpu.VMEM((do, do), jnp.float32),
        ],
    )(x, W1, b1[0, 0][None, :], W2, b2[0, 0][None, :],
      W3, b3[0, 0][None, :], Wc, bc[None, :])
    return (out, h)
